# baseline (speedup 1.0000x reference)
import jax
import jax.numpy as jnp
from jax import lax
from jax.experimental import pallas as pl
from jax.experimental.pallas import tpu as pltpu
from jax.experimental.pallas import tpu_sc as plsc

NUM_SYMBOLS = 100000
EMBED_DIM = 128
NQ = 4096
NS = 128
NW = 32
QPW = NQ // NW
RPW = 2 * QPW
L = 16
NCH = EMBED_DIM // L

_PERM_DN = lax.GatherDimensionNumbers(
    offset_dims=(), collapsed_slice_dims=(0,), start_index_map=(0,))


def _perm(v, idx):
    return lax.gather(v, idx[:, None], dimension_numbers=_PERM_DN,
                      slice_sizes=(1,), mode=lax.GatherScatterMode.PROMISE_IN_BOUNDS)


def _allreduce(v, rots):
    for r in rots:
        v = v + _perm(v, r)
    return v


def _make_kernel():
    mesh = plsc.VectorSubcoreMesh(core_axis_name="c", subcore_axis_name="s")

    def body(query_hbm, support_hbm, table_hbm, out_hbm,
             qidx, sidx, qrows, srows, mbuf, outbuf, sem_s, sem_q):
        nc = 2
        wid = lax.axis_index("s") * nc + lax.axis_index("c")

        pltpu.sync_copy(query_hbm.at[wid], qidx)
        pltpu.sync_copy(support_hbm, sidx)

        cs0 = pltpu.async_copy(table_hbm.at[sidx.at[0]], srows.at[0], sem_s)
        cs1 = pltpu.async_copy(table_hbm.at[sidx.at[1]], srows.at[1], sem_s)
        cqs = [pltpu.async_copy(table_hbm.at[qidx.at[j]], qrows.at[j], sem_q)
               for j in range(4)]
        cs0.wait()
        cs1.wait()

        io16 = lax.iota(jnp.int32, L)
        rots = [(io16 + k) & (L - 1) for k in (8, 4, 2, 1)]
        rot8 = rots[0]
        roth = [(io16 & 8) | ((io16 + k) & 7) for k in (4, 2, 1)]
        lo8 = io16 < 8
        zero = jnp.zeros((L,), jnp.float32)

        accs = (zero,) * (2 * NCH)

        def sbody(bb):
            def f(r, a):
                a = list(a)
                for c in range(NCH):
                    a[c] = a[c] + srows[bb, 2 * r, pl.ds(L * c, L)]
                    a[NCH + c] = a[NCH + c] + srows[bb, 2 * r + 1, pl.ds(L * c, L)]
                return tuple(a)
            return f

        for b in range(2):
            accs = lax.fori_loop(0, 64, sbody(b), accs)
        scale = jnp.float32(1.0 / NS)
        m = [a * scale for a in accs]

        sv = zero
        for c in range(2 * NCH):
            sv = sv + m[c] * m[c]
            mbuf[c, :] = m[c]
        sn2 = _allreduce(sv, rots)

        half = jnp.float32(0.5)
        three_half = jnp.float32(1.5)
        eps = jnp.float32(1e-8)
        one = jnp.float32(1.0)
        tiny = jnp.float32(1e-30)

        def gbody(jj):
            def f(g, carry):
                totn = zero
                totq = zero
                for t in range(4):
                    qs = (t, t + 8, t + 4, t + 12)
                    acc = [zero] * 8
                    for h in range(2):
                        for c in range(NCH):
                            mm = mbuf[NCH * h + c, :]
                            for j, tq in enumerate(qs):
                                v = qrows[jj, 2 * (g * L + tq) + h, pl.ds(L * c, L)]
                                acc[2 * j] = acc[2 * j] + v * mm
                                acc[2 * j + 1] = acc[2 * j + 1] + v * v
                    for p in range(2):
                        na, qa, nb, qb = acc[4 * p:4 * p + 4]
                        cn = jnp.where(lo8, na + _perm(na, rot8), nb + _perm(nb, rot8))
                        cq = jnp.where(lo8, qa + _perm(qa, rot8), qb + _perm(qb, rot8))
                        for rk in roth:
                            cn = cn + _perm(cn, rk)
                            cq = cq + _perm(cq, rk)
                        sel = (io16 & 7) == (t + 4 * p)
                        totn = jnp.where(sel, cn, totn)
                        totq = jnp.where(sel, cq, totq)
                prod = totq * sn2
                y = jnp.minimum(one / jnp.maximum(prod, tiny), one)
                for _ in range(15):
                    y = y * (three_half - half * prod * y * y)
                sq = prod * y
                denom = jnp.maximum(sq, eps)
                outbuf[pl.ds(jj * 32 + g * L, L)] = totn / denom
                return carry
            return f

        for j in range(4):
            cqs[j].wait()
            lax.fori_loop(0, 2, gbody(j), 0)

        pltpu.sync_copy(outbuf, out_hbm.at[pl.ds(wid * QPW, QPW)])

    return pl.kernel(
        body,
        out_type=jax.ShapeDtypeStruct((NQ,), jnp.float32),
        mesh=mesh,
        scratch_types=[
            pltpu.VMEM((4, 64), jnp.int32),
            pltpu.VMEM((2, 128), jnp.int32),
            pltpu.VMEM((4, 64, EMBED_DIM), jnp.float32),
            pltpu.VMEM((2, 128, EMBED_DIM), jnp.float32),
            pltpu.VMEM((L, L), jnp.float32),
            pltpu.VMEM((QPW,), jnp.float32),
            pltpu.SemaphoreType.DMA,
            pltpu.SemaphoreType.DMA,
        ],
    )


_sc_kernel = _make_kernel()


@jax.jit
def kernel(query, support, symbol_emb_weight):
    q = query.astype(jnp.int32).reshape(NW, 4, 64)
    s = support.astype(jnp.int32).reshape(2, 128)
    return _sc_kernel(q, s, symbol_emb_weight)


# speedup vs baseline: 1.0894x; 1.0894x over previous
import jax
import jax.numpy as jnp
from jax import lax
from jax.experimental import pallas as pl
from jax.experimental.pallas import tpu as pltpu
from jax.experimental.pallas import tpu_sc as plsc

NUM_SYMBOLS = 100000
EMBED_DIM = 128
NQ = 4096
NS = 128
NW = 32
QPW = NQ // NW
RPW = 2 * QPW
L = 16
NCH = EMBED_DIM // L

_PERM_DN = lax.GatherDimensionNumbers(
    offset_dims=(), collapsed_slice_dims=(0,), start_index_map=(0,))


def _perm(v, idx):
    return lax.gather(v, idx[:, None], dimension_numbers=_PERM_DN,
                      slice_sizes=(1,), mode=lax.GatherScatterMode.PROMISE_IN_BOUNDS)


def _allreduce(v, rots):
    for r in rots:
        v = v + _perm(v, r)
    return v


def _make_kernel():
    mesh = plsc.VectorSubcoreMesh(core_axis_name="c", subcore_axis_name="s")

    def body(query_hbm, support_hbm, table_hbm, out_hbm,
             qidx, sidx, qrows, srows, mbuf, outbuf, sem_s, sem_q):
        nc = 2
        wid = lax.axis_index("s") * nc + lax.axis_index("c")

        pltpu.sync_copy(query_hbm.at[wid], qidx)
        pltpu.sync_copy(support_hbm, sidx)

        cs0 = pltpu.async_copy(table_hbm.at[sidx.at[0]], srows.at[0], sem_s)
        cs1 = pltpu.async_copy(table_hbm.at[sidx.at[1]], srows.at[1], sem_s)
        cq0 = pltpu.async_copy(table_hbm.at[qidx.at[0]], qrows.at[0], sem_q)
        cq1 = pltpu.async_copy(table_hbm.at[qidx.at[1]], qrows.at[1], sem_q)
        cs0.wait()
        cs1.wait()

        io16 = lax.iota(jnp.int32, L)
        rots = [(io16 + k) & (L - 1) for k in (8, 4, 2, 1)]
        rot8 = rots[0]
        roth = [(io16 & 8) | ((io16 + k) & 7) for k in (4, 2, 1)]
        lo8 = io16 < 8
        zero = jnp.zeros((L,), jnp.float32)

        accs = (zero,) * (2 * NCH)

        def sbody(bb):
            def f(r, a):
                a = list(a)
                for c in range(NCH):
                    a[c] = a[c] + srows[bb, 2 * r, pl.ds(L * c, L)]
                    a[NCH + c] = a[NCH + c] + srows[bb, 2 * r + 1, pl.ds(L * c, L)]
                return tuple(a)
            return f

        for b in range(2):
            accs = lax.fori_loop(0, 64, sbody(b), accs)
        scale = jnp.float32(1.0 / NS)
        m = [a * scale for a in accs]

        sv = zero
        for c in range(2 * NCH):
            sv = sv + m[c] * m[c]
            mbuf[c, :] = m[c]
        sn2 = _allreduce(sv, rots)

        half = jnp.float32(0.5)
        three_half = jnp.float32(1.5)
        eps = jnp.float32(1e-8)
        one = jnp.float32(1.0)
        tiny = jnp.float32(1e-30)

        def gbody(gall, carry):
                bb = gall >> 2
                g = gall & 3
                @pl.when(gall == 4)
                def _():
                    cq1.wait()
                totn = zero
                totq = zero
                for t in range(4):
                    qs = (t, t + 8, t + 4, t + 12)
                    acc = [zero] * 8
                    for h in range(2):
                        for c in range(NCH):
                            mm = mbuf[NCH * h + c, :]
                            for j, tq in enumerate(qs):
                                v = qrows[bb, 2 * (g * L + tq) + h, pl.ds(L * c, L)]
                                acc[2 * j] = acc[2 * j] + v * mm
                                acc[2 * j + 1] = acc[2 * j + 1] + v * v
                    for p in range(2):
                        na, qa, nb, qb = acc[4 * p:4 * p + 4]
                        cn = jnp.where(lo8, na + _perm(na, rot8), nb + _perm(nb, rot8))
                        cq = jnp.where(lo8, qa + _perm(qa, rot8), qb + _perm(qb, rot8))
                        for rk in roth:
                            cn = cn + _perm(cn, rk)
                            cq = cq + _perm(cq, rk)
                        sel = (io16 & 7) == (t + 4 * p)
                        totn = jnp.where(sel, cn, totn)
                        totq = jnp.where(sel, cq, totq)
                prod = totq * sn2
                y = jnp.minimum(one / jnp.maximum(prod, tiny), one)
                for _ in range(15):
                    y = y * (three_half - half * prod * y * y)
                sq = prod * y
                denom = jnp.maximum(sq, eps)
                outbuf[pl.ds(gall * L, L)] = totn / denom
                return carry

        cq0.wait()
        lax.fori_loop(0, 8, gbody, 0)

        pltpu.sync_copy(outbuf, out_hbm.at[pl.ds(wid * QPW, QPW)])

    return pl.kernel(
        body,
        out_type=jax.ShapeDtypeStruct((NQ,), jnp.float32),
        mesh=mesh,
        scratch_types=[
            pltpu.VMEM((2, 128), jnp.int32),
            pltpu.VMEM((2, 128), jnp.int32),
            pltpu.VMEM((2, 128, EMBED_DIM), jnp.float32),
            pltpu.VMEM((2, 128, EMBED_DIM), jnp.float32),
            pltpu.VMEM((L, L), jnp.float32),
            pltpu.VMEM((QPW,), jnp.float32),
            pltpu.SemaphoreType.DMA,
            pltpu.SemaphoreType.DMA,
        ],
    )


_sc_kernel = _make_kernel()


@jax.jit
def kernel(query, support, symbol_emb_weight):
    q = query.astype(jnp.int32).reshape(NW, 2, 128)
    s = support.astype(jnp.int32).reshape(2, 128)
    return _sc_kernel(q, s, symbol_emb_weight)


# speedup vs baseline: 1.1136x; 1.0222x over previous
import jax
import jax.numpy as jnp
from jax import lax
from jax.experimental import pallas as pl
from jax.experimental.pallas import tpu as pltpu
from jax.experimental.pallas import tpu_sc as plsc

NUM_SYMBOLS = 100000
EMBED_DIM = 128
NQ = 4096
NS = 128
NW = 32
QPW = NQ // NW
RPW = 2 * QPW
L = 16
NCH = EMBED_DIM // L

_PERM_DN = lax.GatherDimensionNumbers(
    offset_dims=(), collapsed_slice_dims=(0,), start_index_map=(0,))


def _perm(v, idx):
    return lax.gather(v, idx[:, None], dimension_numbers=_PERM_DN,
                      slice_sizes=(1,), mode=lax.GatherScatterMode.PROMISE_IN_BOUNDS)


def _allreduce(v, rots):
    for r in rots:
        v = v + _perm(v, r)
    return v


def _make_kernel():
    mesh = plsc.VectorSubcoreMesh(core_axis_name="c", subcore_axis_name="s")

    def body(query_hbm, support_hbm, table_hbm, out_hbm,
             qidx, sidx, qrows, srows, mbuf, outbuf, sem_s, sem_q):
        nc = 2
        wid = lax.axis_index("s") * nc + lax.axis_index("c")

        pltpu.sync_copy(query_hbm.at[wid], qidx)
        pltpu.sync_copy(support_hbm, sidx)

        cs0 = pltpu.async_copy(table_hbm.at[sidx.at[0]], srows.at[0], sem_s)
        cs1 = pltpu.async_copy(table_hbm.at[sidx.at[1]], srows.at[1], sem_s)
        cq0 = pltpu.async_copy(table_hbm.at[qidx.at[0]], qrows.at[0], sem_q)
        cq1 = pltpu.async_copy(table_hbm.at[qidx.at[1]], qrows.at[1], sem_q)
        io16 = lax.iota(jnp.int32, L)
        rots = [(io16 + k) & (L - 1) for k in (8, 4, 2, 1)]
        rot8 = rots[0]
        roth = [(io16 & 8) | ((io16 + k) & 7) for k in (4, 2, 1)]
        lo8 = io16 < 8
        zero = jnp.zeros((L,), jnp.float32)

        accs = (zero,) * (2 * NCH)

        def sbody(bb):
            def f(r, a):
                a = list(a)
                for c in range(NCH):
                    a[c] = a[c] + srows[bb, 2 * r, pl.ds(L * c, L)]
                    a[NCH + c] = a[NCH + c] + srows[bb, 2 * r + 1, pl.ds(L * c, L)]
                return tuple(a)
            return f

        cs0.wait()
        accs = lax.fori_loop(0, 64, sbody(0), accs)
        cs1.wait()
        accs = lax.fori_loop(0, 64, sbody(1), accs)
        scale = jnp.float32(1.0 / NS)
        m = [a * scale for a in accs]

        sv = zero
        for c in range(2 * NCH):
            sv = sv + m[c] * m[c]
            mbuf[c, :] = m[c]
        sn2 = _allreduce(sv, rots)

        half = jnp.float32(0.5)
        three_half = jnp.float32(1.5)
        eps = jnp.float32(1e-8)
        one = jnp.float32(1.0)
        tiny = jnp.float32(1e-30)

        def gbody(gall, carry):
                bb = gall >> 2
                g = gall & 3
                @pl.when(gall == 4)
                def _():
                    cq1.wait()
                totn = zero
                totq = zero
                for t in range(4):
                    qs = (t, t + 8, t + 4, t + 12)
                    acc = [zero] * 8
                    for h in range(2):
                        for c in range(NCH):
                            mm = mbuf[NCH * h + c, :]
                            for j, tq in enumerate(qs):
                                v = qrows[bb, 2 * (g * L + tq) + h, pl.ds(L * c, L)]
                                acc[2 * j] = acc[2 * j] + v * mm
                                acc[2 * j + 1] = acc[2 * j + 1] + v * v
                    for p in range(2):
                        na, qa, nb, qb = acc[4 * p:4 * p + 4]
                        cn = jnp.where(lo8, na + _perm(na, rot8), nb + _perm(nb, rot8))
                        cq = jnp.where(lo8, qa + _perm(qa, rot8), qb + _perm(qb, rot8))
                        for rk in roth:
                            cn = cn + _perm(cn, rk)
                            cq = cq + _perm(cq, rk)
                        sel = (io16 & 7) == (t + 4 * p)
                        totn = jnp.where(sel, cn, totn)
                        totq = jnp.where(sel, cq, totq)
                prod = totq * sn2
                y = jnp.minimum(one / jnp.maximum(prod, tiny), one)
                for _ in range(15):
                    y = y * (three_half - half * prod * y * y)
                sq = prod * y
                denom = jnp.maximum(sq, eps)
                outbuf[pl.ds(gall * L, L)] = totn / denom
                return carry

        cq0.wait()
        lax.fori_loop(0, 8, gbody, 0)

        pltpu.sync_copy(outbuf, out_hbm.at[pl.ds(wid * QPW, QPW)])

    return pl.kernel(
        body,
        out_type=jax.ShapeDtypeStruct((NQ,), jnp.float32),
        mesh=mesh,
        scratch_types=[
            pltpu.VMEM((2, 128), jnp.int32),
            pltpu.VMEM((2, 128), jnp.int32),
            pltpu.VMEM((2, 128, EMBED_DIM), jnp.float32),
            pltpu.VMEM((2, 128, EMBED_DIM), jnp.float32),
            pltpu.VMEM((L, L), jnp.float32),
            pltpu.VMEM((QPW,), jnp.float32),
            pltpu.SemaphoreType.DMA,
            pltpu.SemaphoreType.DMA,
        ],
    )


_sc_kernel = _make_kernel()


@jax.jit
def kernel(query, support, symbol_emb_weight):
    q = query.astype(jnp.int32).reshape(NW, 2, 128)
    s = support.astype(jnp.int32).reshape(2, 128)
    return _sc_kernel(q, s, symbol_emb_weight)
